# baseline (speedup 1.0000x reference)
import functools

import jax
import jax.numpy as jnp
import numpy as np
from jax import lax
from jax.experimental import pallas as pl
from jax.experimental.pallas import tpu as pltpu
from jax.experimental.pallas import tpu_sc as plsc

N = 10000
E = 320000
D = 128
NB = 16
H = 128
MAX_R = 5.0

NC = 2
NS = 16
NW = NC * NS
EPW = E // NW
C = 80
NCHUNK = EPW // C
RC = 80
NRC = N // RC
L = 16

_PERM = np.zeros(D, dtype=np.int32)
for _k in range(D // 2):
    _PERM[_k] = 32 * (_k // 16) + (_k % 16)
    _PERM[D // 2 + _k] = 32 * (_k // 16) + 16 + (_k % 16)


def _radial_body(rvt_ref, w1a_ref, w2a_ref, o0_ref):
    x = rvt_ref[0:1, :]
    y = rvt_ref[1:2, :]
    z = rvt_ref[2:3, :]
    r = jnp.sqrt(x * x + y * y + z * z)
    c1 = jnp.cos(r * (jnp.pi / MAX_R))
    mask = (r < MAX_R).astype(jnp.float32)
    rows = [c1 * mask]
    prev2, prev1 = mask, rows[0]
    two_c1 = 2.0 * c1
    for _ in range(NB - 1):
        cur = two_c1 * prev1 - prev2
        rows.append(cur)
        prev2, prev1 = prev1, cur
    basis = jnp.concatenate(rows, axis=0).T
    h0 = jnp.maximum(
        jnp.dot(basis, w1a_ref[...], preferred_element_type=jnp.float32), 0.0)
    rp = jnp.dot(h0, w2a_ref[...], preferred_element_type=jnp.float32)
    ilo = lax.shift_right_logical(
        lax.bitcast_convert_type(rp[:, :D // 2], jnp.int32), 16)
    ihi = lax.bitcast_convert_type(rp[:, D // 2:], jnp.int32) & (-65536)
    o0_ref[...] = ilo | ihi


def _radial(rvt, W1, W2):
    BE = 2560
    grid = E // BE
    wspec = lambda shape: pl.BlockSpec(shape, lambda i: (0, 0))
    return pl.pallas_call(
        _radial_body,
        grid=(grid,),
        in_specs=[
            pl.BlockSpec((3, BE), lambda i: (0, i)),
            wspec((NB, H)), wspec((H, D)),
        ],
        out_specs=pl.BlockSpec((BE, D // 2), lambda i: (i, 0)),
        out_shape=jax.ShapeDtypeStruct((E, D // 2), jnp.int32),
    )(rvt, W1, W2)


def _sc_body(x_hbm, radial_hbm, iab_hbm, out_hbm,
             iab, xr, rr0, rr1,
             si0, si1, si2, si3, sg0, sg1, sr0, sr1, ss0, agg_sh):
    rr = (rr0, rr1)
    c = lax.axis_index("c")
    s = lax.axis_index("s")
    wid = s * NC + c
    si = (si0, si1, si2, si3)
    sg = (sg0, sg1)
    sr = (sr0, sr1)

    zero16 = jnp.zeros((L,), jnp.float32)

    def _zrow(i, _):
        for j in range(D // L):
            xr[0, i, pl.ds(j * L, L)] = zero16
        return 0

    lax.fori_loop(0, RC, _zrow, 0)
    for k in range(-(-NRC // NS)):
        t = s + k * NS

        @pl.when(t < NRC)
        def _():
            r0 = pl.multiple_of(t * RC, RC)
            pltpu.sync_copy(xr.at[0], agg_sh.at[pl.ds(r0, RC), :])

    plsc.subcore_barrier()

    ebase = pl.multiple_of(wid * EPW, EPW)

    def _fetch_idx(t, q):
        pltpu.async_copy(iab_hbm.at[wid, t], iab.at[q], si[q])

    def _wait_idx(t, q):
        pltpu.make_async_copy(
            iab_hbm.at[wid, t], iab.at[q], si[q]).wait()

    def _issue(t, q, p):
        pltpu.async_copy(x_hbm.at[iab.at[q, 1]], xr.at[p], sg[p])
        base = pl.multiple_of((ebase + t * C) * (D // 2), C * D // 2)
        pltpu.async_copy(radial_hbm.at[pl.ds(base, C * D // 2)], rr[p], sr[p])

    def _wait_gr(t, q, p):
        pltpu.make_async_copy(x_hbm.at[iab.at[q, 1]], xr.at[p], sg[p]).wait()
        base = pl.multiple_of((ebase + t * C) * (D // 2), C * D // 2)
        pltpu.make_async_copy(
            radial_hbm.at[pl.ds(base, C * D // 2)], rr[p], sr[p]).wait()

    himask = jnp.full((L,), -65536, dtype=jnp.int32)
    sh16 = jnp.full((L,), 16, dtype=jnp.int32)

    def _mul(p):
        def _mrow(i, _):
            roff = pl.multiple_of(i * (D // 2), D // 2)
            for j in range(D // 32):
                rbits = rr[p][pl.ds(roff + j * L, L)]
                flo = lax.bitcast_convert_type(
                    lax.shift_left(rbits, sh16), jnp.float32)
                fhi = lax.bitcast_convert_type(rbits & himask, jnp.float32)
                slo = pl.ds(j * 32, L)
                shi = pl.ds(j * 32 + L, L)
                xr[p, i, slo] = xr[p, i, slo] * flo
                xr[p, i, shi] = xr[p, i, shi] * fhi
            return 0

        lax.fori_loop(0, C, _mrow, 0)

    def _drain_scatter(p):
        pltpu.make_async_copy(x_hbm.at[pl.ds(0, C), :], xr.at[p], ss0).wait()

    def _scatter(q, p):
        pltpu.async_copy(xr.at[p], agg_sh.at[iab.at[q, 0]], ss0, add=True)

    _fetch_idx(0, 0)
    _fetch_idx(1, 1)
    _wait_idx(0, 0)
    _issue(0, 0, 0)

    def _quad(g, _):
        for u in range(4):
            t = 4 * g + u
            p = u % 2
            _wait_idx(t + 1, (u + 1) % 4)

            def _w():
                _drain_scatter(1 - p)

            if u == 0:
                pl.when(g > 0)(_w)
            else:
                _w()
            _issue(t + 1, (u + 1) % 4, 1 - p)
            _wait_gr(t, u, p)
            _mul(p)

            def _f():
                _fetch_idx(t + 2, (u + 2) % 4)

            if u == 3:
                pl.when(g < (NCHUNK - 1) // 4 - 1)(_f)
            else:
                _f()
            _scatter(u, p)
        return 0

    lax.fori_loop(0, (NCHUNK - 1) // 4, _quad, 0)

    tl = NCHUNK - 1
    _wait_gr(tl, 0, 0)
    _drain_scatter(1)
    _mul(0)
    pltpu.sync_copy(xr.at[0], agg_sh.at[iab.at[0, 0]], add=True)
    plsc.subcore_barrier()

    for k in range(-(-NRC // NS)):
        t = s + k * NS

        @pl.when(t < NRC)
        def _():
            r0 = pl.multiple_of(t * RC, RC)
            pltpu.sync_copy(agg_sh.at[pl.ds(r0, RC), :], xr.at[0])
            pltpu.sync_copy(xr.at[0], out_hbm.at[c, pl.ds(r0, RC), :])


@functools.cache
def _make_sc_layer():
    return pl.kernel(
        _sc_body,
        out_type=jax.ShapeDtypeStruct((NC, N, D), jnp.float32),
        mesh=plsc.VectorSubcoreMesh(
            core_axis_name="c", subcore_axis_name="s",
            num_cores=NC, num_subcores=NS),
        scratch_types=[
            pltpu.VMEM((4, 2, C), jnp.int32),
            pltpu.VMEM((2, C, D), jnp.float32),
            pltpu.VMEM((C * D // 2,), jnp.int32),
            pltpu.VMEM((C * D // 2,), jnp.int32),
        ] + [pltpu.SemaphoreType.DMA] * 9 + [
            pltpu.VMEM_SHARED((N, D), jnp.float32),
        ],
    )


def _sc_layer(x, radial, iab4):
    return _make_sc_layer()(x, radial.reshape(E * D // 2), iab4)


def _mix_body(p_ref, ws_ref, o_ref):
    agg = p_ref[0] + p_ref[1]
    y = jnp.dot(agg, ws_ref[...], preferred_element_type=jnp.float32)
    o_ref[...] = y * jax.nn.sigmoid(y)


def _mix(partial, Ws_scaled):
    BN = 2000
    grid = N // BN
    return pl.pallas_call(
        _mix_body,
        grid=(grid,),
        in_specs=[
            pl.BlockSpec((NC, BN, D), lambda i: (0, i, 0)),
            pl.BlockSpec((D, D), lambda i: (0, 0)),
        ],
        out_specs=pl.BlockSpec((BN, D), lambda i: (i, 0)),
        out_shape=jax.ShapeDtypeStruct((N, D), jnp.float32),
    )(partial, Ws_scaled)


def _mix_mean_body(p_ref, ws_ref, o_ref):
    i = pl.program_id(0)
    agg = p_ref[0] + p_ref[1]
    y = jnp.dot(agg, ws_ref[...], preferred_element_type=jnp.float32)
    g = y * jax.nn.sigmoid(y)
    part = jnp.sum(g, axis=0, keepdims=True) * (1.0 / N)

    @pl.when(i == 0)
    def _():
        o_ref[...] = jnp.zeros_like(o_ref)

    o_ref[...] += part


def _mix_mean(partial, Ws_scaled):
    BN = 2000
    grid = N // BN
    out = pl.pallas_call(
        _mix_mean_body,
        grid=(grid,),
        in_specs=[
            pl.BlockSpec((NC, BN, D), lambda i: (0, i, 0)),
            pl.BlockSpec((D, D), lambda i: (0, 0)),
        ],
        out_specs=pl.BlockSpec((1, D), lambda i: (0, 0)),
        out_shape=jax.ShapeDtypeStruct((1, D), jnp.float32),
    )(partial, Ws_scaled)
    return out[0]


def kernel(features, radii_vectors, W1_0, W2_0, Ws_0, W1_1, W2_1, Ws_1,
           n_norm, ab_p_to_a, ab_p_to_b):
    inv_norm = 1.0 / jnp.sqrt(jnp.asarray(n_norm, dtype=jnp.float32))
    iab4 = jnp.stack([ab_p_to_a.reshape(NW, NCHUNK, C),
                      ab_p_to_b.reshape(NW, NCHUNK, C)], axis=2)
    rvt = radii_vectors.T
    radial0 = _radial(rvt, W1_0, W2_0[:, _PERM])
    p0 = _sc_layer(features, radial0, iab4)
    radial1 = _radial(rvt, W1_1, W2_1[:, _PERM])
    x1 = _mix(p0, Ws_0 * inv_norm)
    p1 = _sc_layer(x1, radial1, iab4)
    return _mix_mean(p1, Ws_1 * inv_norm)


# speedup vs baseline: 1.5958x; 1.5958x over previous
import functools

import jax
import jax.numpy as jnp
from jax import lax
from jax.experimental import pallas as pl
from jax.experimental.pallas import tpu as pltpu
from jax.experimental.pallas import tpu_sc as plsc

N = 10000
E = 320000
D = 128
NB = 16
H = 128
MAX_R = 5.0

NC = 2
NS = 16
NW = NC * NS
EPW = E // NW
C = 80
NCHUNK = EPW // C
RC = 80
NRC = N // RC
L = 16


def _radial_body(rvt_ref, w1a_ref, w2a_ref, o0_ref):
    x = rvt_ref[0:1, :]
    y = rvt_ref[1:2, :]
    z = rvt_ref[2:3, :]
    r = jnp.sqrt(x * x + y * y + z * z)
    c1 = jnp.cos(r * (jnp.pi / MAX_R))
    mask = (r < MAX_R).astype(jnp.float32)
    rows = [c1 * mask]
    prev2, prev1 = mask, rows[0]
    two_c1 = 2.0 * c1
    for _ in range(NB - 1):
        cur = two_c1 * prev1 - prev2
        rows.append(cur)
        prev2, prev1 = prev1, cur
    basis = jnp.concatenate(rows, axis=0).T
    h0 = jnp.maximum(
        jnp.dot(basis, w1a_ref[...], preferred_element_type=jnp.float32), 0.0)
    o0_ref[...] = jnp.dot(h0, w2a_ref[...], preferred_element_type=jnp.float32)


def _radial(rvt, W1, W2):
    BE = 2560
    grid = E // BE
    wspec = lambda shape: pl.BlockSpec(shape, lambda i: (0, 0))
    return pl.pallas_call(
        _radial_body,
        grid=(grid,),
        in_specs=[
            pl.BlockSpec((3, BE), lambda i: (0, i)),
            wspec((NB, H)), wspec((H, D)),
        ],
        out_specs=pl.BlockSpec((BE, D), lambda i: (i, 0)),
        out_shape=jax.ShapeDtypeStruct((E, D), jnp.float32),
    )(rvt, W1, W2)


def _sc_body(x_hbm, radial_hbm, iab_hbm, out_hbm,
             iab, xr, rr,
             si0, si1, si2, si3, sg0, sg1, sr0, sr1, ss0, agg_sh):
    c = lax.axis_index("c")
    s = lax.axis_index("s")
    wid = s * NC + c
    si = (si0, si1, si2, si3)
    sg = (sg0, sg1)
    sr = (sr0, sr1)

    zero16 = jnp.zeros((L,), jnp.float32)

    def _zrow(i, _):
        for j in range(D // L):
            xr[0, i, pl.ds(j * L, L)] = zero16
        return 0

    lax.fori_loop(0, RC, _zrow, 0)
    for k in range(-(-NRC // NS)):
        t = s + k * NS

        @pl.when(t < NRC)
        def _():
            r0 = pl.multiple_of(t * RC, RC)
            pltpu.sync_copy(xr.at[0], agg_sh.at[pl.ds(r0, RC), :])

    plsc.subcore_barrier()

    ebase = pl.multiple_of(wid * EPW, EPW)

    def _fetch_idx(t, q):
        pltpu.async_copy(iab_hbm.at[wid, t], iab.at[q], si[q])

    def _wait_idx(t, q):
        pltpu.make_async_copy(
            iab_hbm.at[wid, t], iab.at[q], si[q]).wait()

    def _issue(t, q, p):
        pltpu.async_copy(x_hbm.at[iab.at[q, 1]], xr.at[p], sg[p])
        base = pl.multiple_of(ebase + t * C, C)
        pltpu.async_copy(radial_hbm.at[pl.ds(base, C), :], rr.at[p], sr[p])

    def _wait_gr(t, q, p):
        pltpu.make_async_copy(x_hbm.at[iab.at[q, 1]], xr.at[p], sg[p]).wait()
        base = pl.multiple_of(ebase + t * C, C)
        pltpu.make_async_copy(
            radial_hbm.at[pl.ds(base, C), :], rr.at[p], sr[p]).wait()

    def _mul(p):
        def _mrow(i, _):
            for j in range(D // L):
                sl = pl.ds(j * L, L)
                xr[p, i, sl] = xr[p, i, sl] * rr[p, i, sl]
            return 0

        lax.fori_loop(0, C, _mrow, 0)

    def _drain_scatter(p):
        pltpu.make_async_copy(x_hbm.at[pl.ds(0, C), :], xr.at[p], ss0).wait()

    def _scatter(q, p):
        pltpu.async_copy(xr.at[p], agg_sh.at[iab.at[q, 0]], ss0, add=True)

    _fetch_idx(0, 0)
    _fetch_idx(1, 1)
    _wait_idx(0, 0)
    _issue(0, 0, 0)

    def _quad(g, _):
        for u in range(4):
            t = 4 * g + u
            p = u % 2
            _wait_idx(t + 1, (u + 1) % 4)

            def _w():
                _drain_scatter(1 - p)

            if u == 0:
                pl.when(g > 0)(_w)
            else:
                _w()
            _issue(t + 1, (u + 1) % 4, 1 - p)
            _wait_gr(t, u, p)
            _mul(p)

            def _f():
                _fetch_idx(t + 2, (u + 2) % 4)

            if u == 3:
                pl.when(g < (NCHUNK - 1) // 4 - 1)(_f)
            else:
                _f()
            _scatter(u, p)
        return 0

    lax.fori_loop(0, (NCHUNK - 1) // 4, _quad, 0)

    tl = NCHUNK - 1
    _wait_gr(tl, 0, 0)
    _drain_scatter(1)
    _mul(0)
    pltpu.sync_copy(xr.at[0], agg_sh.at[iab.at[0, 0]], add=True)
    plsc.subcore_barrier()

    for k in range(-(-NRC // NS)):
        t = s + k * NS

        @pl.when(t < NRC)
        def _():
            r0 = pl.multiple_of(t * RC, RC)
            pltpu.sync_copy(agg_sh.at[pl.ds(r0, RC), :], xr.at[0])
            pltpu.sync_copy(xr.at[0], out_hbm.at[c, pl.ds(r0, RC), :])


@functools.cache
def _make_sc_layer():
    return pl.kernel(
        _sc_body,
        out_type=jax.ShapeDtypeStruct((NC, N, D), jnp.float32),
        mesh=plsc.VectorSubcoreMesh(
            core_axis_name="c", subcore_axis_name="s",
            num_cores=NC, num_subcores=NS),
        scratch_types=[
            pltpu.VMEM((4, 2, C), jnp.int32),
            pltpu.VMEM((2, C, D), jnp.float32),
            pltpu.VMEM((2, C, D), jnp.float32),
        ] + [pltpu.SemaphoreType.DMA] * 9 + [
            pltpu.VMEM_SHARED((N, D), jnp.float32),
        ],
    )


def _sc_layer(x, radial, iab4):
    return _make_sc_layer()(x, radial, iab4)


def _mix_body(p_ref, ws_ref, o_ref):
    agg = p_ref[0] + p_ref[1]
    y = jnp.dot(agg, ws_ref[...], preferred_element_type=jnp.float32)
    o_ref[...] = y * jax.nn.sigmoid(y)


def _mix(partial, Ws_scaled):
    BN = 2000
    grid = N // BN
    return pl.pallas_call(
        _mix_body,
        grid=(grid,),
        in_specs=[
            pl.BlockSpec((NC, BN, D), lambda i: (0, i, 0)),
            pl.BlockSpec((D, D), lambda i: (0, 0)),
        ],
        out_specs=pl.BlockSpec((BN, D), lambda i: (i, 0)),
        out_shape=jax.ShapeDtypeStruct((N, D), jnp.float32),
    )(partial, Ws_scaled)


def _mix_mean_body(p_ref, ws_ref, o_ref):
    i = pl.program_id(0)
    agg = p_ref[0] + p_ref[1]
    y = jnp.dot(agg, ws_ref[...], preferred_element_type=jnp.float32)
    g = y * jax.nn.sigmoid(y)
    part = jnp.sum(g, axis=0, keepdims=True) * (1.0 / N)

    @pl.when(i == 0)
    def _():
        o_ref[...] = jnp.zeros_like(o_ref)

    o_ref[...] += part


def _mix_mean(partial, Ws_scaled):
    BN = 2000
    grid = N // BN
    out = pl.pallas_call(
        _mix_mean_body,
        grid=(grid,),
        in_specs=[
            pl.BlockSpec((NC, BN, D), lambda i: (0, i, 0)),
            pl.BlockSpec((D, D), lambda i: (0, 0)),
        ],
        out_specs=pl.BlockSpec((1, D), lambda i: (0, 0)),
        out_shape=jax.ShapeDtypeStruct((1, D), jnp.float32),
    )(partial, Ws_scaled)
    return out[0]


def kernel(features, radii_vectors, W1_0, W2_0, Ws_0, W1_1, W2_1, Ws_1,
           n_norm, ab_p_to_a, ab_p_to_b):
    inv_norm = 1.0 / jnp.sqrt(jnp.asarray(n_norm, dtype=jnp.float32))
    iab4 = jnp.stack([ab_p_to_a.reshape(NW, NCHUNK, C),
                      ab_p_to_b.reshape(NW, NCHUNK, C)], axis=2)
    rvt = radii_vectors.T
    radial0 = _radial(rvt, W1_0, W2_0)
    p0 = _sc_layer(features, radial0, iab4)
    radial1 = _radial(rvt, W1_1, W2_1)
    x1 = _mix(p0, Ws_0 * inv_norm)
    p1 = _sc_layer(x1, radial1, iab4)
    return _mix_mean(p1, Ws_1 * inv_norm)
